# baseline (speedup 1.0000x reference)
import functools

import jax
import jax.numpy as jnp
from jax import lax
from jax.experimental import pallas as pl
from jax.experimental.pallas import tpu as pltpu
from jax.experimental.pallas import tpu_sc as plsc

_NC = 2
_NS = 16
_NW = _NC * _NS
_K = 128
_D = 64
_V = 7
_NBUF = 4


def _gather_body(nchunks, ids_hbm, table_hbm, out_hbm, idx_v, table_v,
                 *bufs_and_sems):
    rows = bufs_and_sems[:_NBUF]
    gsem = bufs_and_sems[_NBUF:2 * _NBUF]
    wsem = bufs_and_sems[2 * _NBUF:3 * _NBUF]
    wid = lax.axis_index("s") * _NC + lax.axis_index("c")
    pltpu.sync_copy(table_hbm, table_v)
    pltpu.sync_copy(ids_hbm.at[pl.ds(wid * nchunks, nchunks)], idx_v)
    row_base = wid * nchunks * _K

    def gather_start(c, b):
        pltpu.async_copy(table_v.at[idx_v.at[c]], rows[b], gsem[b])

    def gather_wait(b):
        pltpu.make_async_copy(table_v.at[idx_v.at[0]], rows[b],
                              gsem[b]).wait()

    def write_start(c, b):
        pltpu.async_copy(rows[b],
                         out_hbm.at[pl.ds(row_base + c * _K, _K)], wsem[b])

    def write_wait(b):
        pltpu.make_async_copy(rows[b], out_hbm.at[pl.ds(row_base, _K)],
                              wsem[b]).wait()

    for c0 in range(_NBUF - 1):
        gather_start(c0, c0)

    def step(cc, carry):
        for b in range(_NBUF):
            c = cc * _NBUF + b
            gather_wait(b)
            nxt = c + _NBUF - 1
            nb = (b + _NBUF - 1) % _NBUF

            @pl.when(nxt < nchunks)
            def _():
                @pl.when(c >= 1)
                def _():
                    write_wait(nb)

                gather_start(nxt, nb)

            write_start(c, b)
        return carry

    lax.fori_loop(0, nchunks // _NBUF, step, 0)
    for b in range(_NBUF):
        write_wait(b)


def kernel(X, color_table, pos_table):
    del pos_table
    batch, seq = X.shape
    nb = seq // 5
    xr = X.reshape(batch, nb, 5).astype(jnp.int32)
    pids = (xr[:, :, 1:5:2] * _V + xr[:, :, 2:5:2]).reshape(-1)
    pair_table = jnp.concatenate(
        [
            jnp.broadcast_to(color_table[:, None, :], (_V, _V, _D)),
            jnp.broadcast_to(color_table[None, :, :], (_V, _V, _D)),
        ],
        axis=-1,
    ).reshape(_V * _V, 2 * _D)

    g = pids.shape[0]
    nchunks = g // (_NW * _K)
    ids2d = pids.reshape(_NW * nchunks, _K)

    body = functools.partial(_gather_body, nchunks)
    out2d = pl.kernel(
        body,
        out_type=jax.ShapeDtypeStruct((g, 2 * _D), jnp.float32),
        mesh=plsc.VectorSubcoreMesh(core_axis_name="c", subcore_axis_name="s"),
        scratch_types=(
            [pltpu.VMEM((nchunks, _K), jnp.int32),
             pltpu.VMEM_SHARED((_V * _V, 2 * _D), jnp.float32)]
            + [pltpu.VMEM((_K, 2 * _D), jnp.float32)] * _NBUF
            + [pltpu.SemaphoreType.DMA] * (2 * _NBUF)
        ),
    )(ids2d, pair_table)
    return out2d.reshape(batch, nb * 4 * _D)


# speedup vs baseline: 3.8079x; 3.8079x over previous
import functools

import jax
import jax.numpy as jnp
from jax import lax
from jax.experimental import pallas as pl
from jax.experimental.pallas import tpu as pltpu
from jax.experimental.pallas import tpu_sc as plsc

_NC = 2
_NS = 16
_NW = _NC * _NS
_K = 128
_D = 64
_V = 7
_NBL = 14
_NBUF = 4

_COLS = tuple((j // 2) * 5 + 1 + 2 * (j % 2) for j in range(_NBL))


def _gather_body(spw, xt_hbm, table_hbm, out_hbm, xcols, idx_v, table_v,
                 *bufs_and_sems):
    nsub = spw // _K
    nchunks = _NBL * nsub
    rows = bufs_and_sems[:_NBUF]
    gsem = bufs_and_sems[_NBUF:2 * _NBUF]
    wsem = bufs_and_sems[2 * _NBUF:3 * _NBUF]
    ssem = bufs_and_sems[3 * _NBUF]
    wid = lax.axis_index("s") * _NC + lax.axis_index("c")
    samp0 = wid * spw
    pltpu.sync_copy(table_hbm, table_v)
    for j, col in enumerate(_COLS):
        pltpu.async_copy(xt_hbm.at[pl.ds(col, 1), pl.ds(samp0, spw)],
                         xcols.at[j, 0], ssem)
        pltpu.async_copy(xt_hbm.at[pl.ds(col + 1, 1), pl.ds(samp0, spw)],
                         xcols.at[j, 1], ssem)
    for j in range(_NBL):
        pltpu.make_async_copy(xt_hbm.at[pl.ds(0, 1), pl.ds(0, spw)],
                              xcols.at[j, 0], ssem).wait()
        pltpu.make_async_copy(xt_hbm.at[pl.ds(0, 1), pl.ds(0, spw)],
                              xcols.at[j, 1], ssem).wait()

    v7 = jnp.full((16,), _V, jnp.int32)
    for j in range(_NBL):
        def pid_step(s, carry, j=j):
            for k2 in range(_K // 16):
                o = s * _K + k2 * 16
                xa = xcols[j, 0, 0, pl.ds(o, 16)]
                xb = xcols[j, 1, 0, pl.ds(o, 16)]
                idx_v[j, s, pl.ds(k2 * 16, 16)] = xa * v7 + xb
            return carry

        lax.fori_loop(0, nsub, pid_step, 0)

    def gather_start(c, b):
        pltpu.async_copy(table_v.at[idx_v.at[c // nsub, c % nsub]],
                         rows[b], gsem[b])

    def gather_wait(b):
        pltpu.make_async_copy(table_v.at[idx_v.at[0, 0]], rows[b],
                              gsem[b]).wait()

    def write_start(c, b):
        pltpu.async_copy(
            rows[b],
            out_hbm.at[pl.ds(samp0 + (c % nsub) * _K, _K),
                       pl.ds((c // nsub) * _K, _K)],
            wsem[b])

    def write_wait(b):
        pltpu.make_async_copy(rows[b],
                              out_hbm.at[pl.ds(0, _K), pl.ds(0, _K)],
                              wsem[b]).wait()

    for c0 in range(_NBUF - 1):
        gather_start(c0, c0)

    def step(cc, carry):
        for b in range(_NBUF):
            c = cc * _NBUF + b
            gather_wait(b)
            nxt = c + _NBUF - 1
            nb = (b + _NBUF - 1) % _NBUF

            @pl.when(nxt < nchunks)
            def _():
                @pl.when(c >= 1)
                def _():
                    write_wait(nb)

                gather_start(nxt, nb)

            write_start(c, b)
        return carry

    lax.fori_loop(0, nchunks // _NBUF, step, 0)
    for b in range(_NBUF):
        write_wait(b)


def kernel(X, color_table, pos_table):
    del pos_table
    batch, seq = X.shape
    nb = seq // 5
    pair_table = jnp.concatenate(
        [
            jnp.broadcast_to(color_table[:, None, :], (_V, _V, _D)),
            jnp.broadcast_to(color_table[None, :, :], (_V, _V, _D)),
        ],
        axis=-1,
    ).reshape(_V * _V, 2 * _D)

    spw = batch // _NW
    xt = jnp.transpose(X.astype(jnp.int32))

    body = functools.partial(_gather_body, spw)
    return pl.kernel(
        body,
        out_type=jax.ShapeDtypeStruct((batch, nb * 4 * _D), jnp.float32),
        mesh=plsc.VectorSubcoreMesh(core_axis_name="c", subcore_axis_name="s"),
        scratch_types=(
            [pltpu.VMEM((_NBL, 2, 1, spw), jnp.int32),
             pltpu.VMEM((_NBL, spw // _K, _K), jnp.int32),
             pltpu.VMEM_SHARED((_V * _V, 2 * _D), jnp.float32)]
            + [pltpu.VMEM((_K, 2 * _D), jnp.float32)] * _NBUF
            + [pltpu.SemaphoreType.DMA] * (2 * _NBUF + 1)
        ),
    )(xt, pair_table)


# speedup vs baseline: 4.0011x; 1.0507x over previous
import functools

import jax
import jax.numpy as jnp
from jax import lax
from jax.experimental import pallas as pl
from jax.experimental.pallas import tpu as pltpu
from jax.experimental.pallas import tpu_sc as plsc

_NC = 2
_NS = 16
_NW = _NC * _NS
_K = 128
_D = 64
_V = 7
_NBL = 14
_NBUF = 4

_COLS = tuple((j // 2) * 5 + 1 + 2 * (j % 2) for j in range(_NBL))


def _gather_body(spw, xt_hbm, table_hbm, out_hbm, xcols, idx_v, table_v,
                 *bufs_and_sems):
    nsub = spw // _K
    nchunks = _NBL * nsub
    rows = bufs_and_sems[:_NBUF]
    gsem = bufs_and_sems[_NBUF:2 * _NBUF]
    wsem = bufs_and_sems[2 * _NBUF:3 * _NBUF]
    ssem = bufs_and_sems[3 * _NBUF]
    wid = lax.axis_index("s") * _NC + lax.axis_index("c")
    samp0 = wid * spw
    pltpu.sync_copy(table_hbm, table_v)
    for j, col in enumerate(_COLS):
        pltpu.async_copy(xt_hbm.at[pl.ds(col, 1), pl.ds(samp0, spw)],
                         xcols.at[j, 0], ssem)
        pltpu.async_copy(xt_hbm.at[pl.ds(col + 1, 1), pl.ds(samp0, spw)],
                         xcols.at[j, 1], ssem)
    for j in range(_NBL):
        pltpu.make_async_copy(xt_hbm.at[pl.ds(0, 1), pl.ds(0, spw)],
                              xcols.at[j, 0], ssem).wait()
        pltpu.make_async_copy(xt_hbm.at[pl.ds(0, 1), pl.ds(0, spw)],
                              xcols.at[j, 1], ssem).wait()

    v7 = jnp.full((16,), _V, jnp.int32)

    def compute_idx(c):
        j = c // nsub
        s = c % nsub
        for k2 in range(_K // 16):
            o = s * _K + k2 * 16
            xa = xcols[j, 0, 0, pl.ds(o, 16)]
            xb = xcols[j, 1, 0, pl.ds(o, 16)]
            idx_v[j, s, pl.ds(k2 * 16, 16)] = xa * v7 + xb

    def gather_start(c, b):
        pltpu.async_copy(table_v.at[idx_v.at[c // nsub, c % nsub]],
                         rows[b], gsem[b])

    def gather_wait(b):
        pltpu.make_async_copy(table_v.at[idx_v.at[0, 0]], rows[b],
                              gsem[b]).wait()

    def write_start(c, b):
        pltpu.async_copy(
            rows[b],
            out_hbm.at[pl.ds(samp0 + (c % nsub) * _K, _K),
                       pl.ds((c // nsub) * _K, _K)],
            wsem[b])

    def write_wait(b):
        pltpu.make_async_copy(rows[b],
                              out_hbm.at[pl.ds(0, _K), pl.ds(0, _K)],
                              wsem[b]).wait()

    for c0 in range(_NBUF - 1):
        compute_idx(c0)
        gather_start(c0, c0)

    def step(cc, carry):
        for b in range(_NBUF):
            c = cc * _NBUF + b
            gather_wait(b)
            nxt = c + _NBUF - 1
            nb = (b + _NBUF - 1) % _NBUF

            @pl.when(nxt < nchunks)
            def _():
                compute_idx(nxt)

                @pl.when(c >= 1)
                def _():
                    write_wait(nb)

                gather_start(nxt, nb)

            write_start(c, b)
        return carry

    lax.fori_loop(0, nchunks // _NBUF, step, 0)
    for b in range(_NBUF):
        write_wait(b)


def kernel(X, color_table, pos_table):
    del pos_table
    batch, seq = X.shape
    nb = seq // 5
    pair_table = jnp.concatenate(
        [
            jnp.broadcast_to(color_table[:, None, :], (_V, _V, _D)),
            jnp.broadcast_to(color_table[None, :, :], (_V, _V, _D)),
        ],
        axis=-1,
    ).reshape(_V * _V, 2 * _D)

    spw = batch // _NW
    xt = jnp.transpose(X.astype(jnp.int32))

    body = functools.partial(_gather_body, spw)
    return pl.kernel(
        body,
        out_type=jax.ShapeDtypeStruct((batch, nb * 4 * _D), jnp.float32),
        mesh=plsc.VectorSubcoreMesh(core_axis_name="c", subcore_axis_name="s"),
        scratch_types=(
            [pltpu.VMEM((_NBL, 2, 1, spw), jnp.int32),
             pltpu.VMEM((_NBL, spw // _K, _K), jnp.int32),
             pltpu.VMEM_SHARED((_V * _V, 2 * _D), jnp.float32)]
            + [pltpu.VMEM((_K, 2 * _D), jnp.float32)] * _NBUF
            + [pltpu.SemaphoreType.DMA] * (2 * _NBUF + 1)
        ),
    )(xt, pair_table)


# speedup vs baseline: 4.0338x; 1.0082x over previous
import functools

import jax
import jax.numpy as jnp
from jax import lax
from jax.experimental import pallas as pl
from jax.experimental.pallas import tpu as pltpu
from jax.experimental.pallas import tpu_sc as plsc

_NC = 2
_NS = 16
_NW = _NC * _NS
_K = 128
_D = 64
_V = 7
_NBL = 14
_SEQ = 35
_NBUF = 4


def _gather_body(spw, xt_hbm, table_hbm, out_hbm, xcols, idx_v, table_v,
                 *bufs_and_sems):
    nsub = spw // _K
    nchunks = _NBL * nsub
    rows = bufs_and_sems[:_NBUF]
    gsem = bufs_and_sems[_NBUF:2 * _NBUF]
    wsem = bufs_and_sems[2 * _NBUF:3 * _NBUF]
    ssem = bufs_and_sems[3 * _NBUF]
    wid = lax.axis_index("s") * _NC + lax.axis_index("c")
    samp0 = wid * spw
    pltpu.async_copy(table_hbm, table_v, ssem)
    pltpu.async_copy(xt_hbm.at[pl.ds(0, _SEQ), pl.ds(samp0, spw)],
                     xcols, ssem)
    pltpu.make_async_copy(table_hbm, table_v, ssem).wait()
    pltpu.make_async_copy(xt_hbm.at[pl.ds(0, _SEQ), pl.ds(0, spw)],
                          xcols, ssem).wait()

    v7 = jnp.full((16,), _V, jnp.int32)

    def compute_idx(c):
        j = c // nsub
        s = c % nsub
        col = (j >> 1) * 5 + 1 + 2 * (j & 1)
        for k2 in range(_K // 16):
            o = s * _K + k2 * 16
            xa = xcols[col, pl.ds(o, 16)]
            xb = xcols[col + 1, pl.ds(o, 16)]
            idx_v[j, s, pl.ds(k2 * 16, 16)] = xa * v7 + xb

    def gather_start(c, b):
        pltpu.async_copy(table_v.at[idx_v.at[c // nsub, c % nsub]],
                         rows[b], gsem[b])

    def gather_wait(b):
        pltpu.make_async_copy(table_v.at[idx_v.at[0, 0]], rows[b],
                              gsem[b]).wait()

    def write_start(c, b):
        pltpu.async_copy(
            rows[b],
            out_hbm.at[pl.ds(samp0 + (c % nsub) * _K, _K),
                       pl.ds((c // nsub) * _K, _K)],
            wsem[b])

    def write_wait(b):
        pltpu.make_async_copy(rows[b],
                              out_hbm.at[pl.ds(0, _K), pl.ds(0, _K)],
                              wsem[b]).wait()

    for c0 in range(_NBUF - 1):
        compute_idx(c0)
        gather_start(c0, c0)

    def step(cc, carry):
        for b in range(_NBUF):
            c = cc * _NBUF + b
            gather_wait(b)
            nxt = c + _NBUF - 1
            nb = (b + _NBUF - 1) % _NBUF

            @pl.when(nxt < nchunks)
            def _():
                compute_idx(nxt)

                @pl.when(c >= 1)
                def _():
                    write_wait(nb)

                gather_start(nxt, nb)

            write_start(c, b)
        return carry

    lax.fori_loop(0, nchunks // _NBUF, step, 0)
    for b in range(_NBUF):
        write_wait(b)


def kernel(X, color_table, pos_table):
    del pos_table
    batch, seq = X.shape
    nb = seq // 5
    pair_table = jnp.concatenate(
        [
            jnp.broadcast_to(color_table[:, None, :], (_V, _V, _D)),
            jnp.broadcast_to(color_table[None, :, :], (_V, _V, _D)),
        ],
        axis=-1,
    ).reshape(_V * _V, 2 * _D)

    spw = batch // _NW
    xt = jnp.transpose(X.astype(jnp.int32))

    body = functools.partial(_gather_body, spw)
    return pl.kernel(
        body,
        out_type=jax.ShapeDtypeStruct((batch, nb * 4 * _D), jnp.float32),
        mesh=plsc.VectorSubcoreMesh(core_axis_name="c", subcore_axis_name="s"),
        scratch_types=(
            [pltpu.VMEM((_SEQ, spw), jnp.int32),
             pltpu.VMEM((_NBL, spw // _K, _K), jnp.int32),
             pltpu.VMEM_SHARED((_V * _V, 2 * _D), jnp.float32)]
            + [pltpu.VMEM((_K, 2 * _D), jnp.float32)] * _NBUF
            + [pltpu.SemaphoreType.DMA] * (2 * _NBUF + 1)
        ),
    )(xt, pair_table)
